# Initial kernel scaffold; baseline (speedup 1.0000x reference)
import jax
import jax.numpy as jnp
from jax.experimental import pallas as pl


def kernel(class_logits, pred_boxes, targets, sizes):
    raise NotImplementedError("write your pallas kernel here")


# speedup vs baseline: 2.9313x; 2.9313x over previous
import functools

import jax
import jax.numpy as jnp
from jax import lax
from jax.experimental import pallas as pl
from jax.experimental.pallas import tpu as pltpu

EOS_COEF = 0.1


def _loss_body(nb, s, num_classes, logits_ref, boxes_ref, tgt_ref, sizes_ref,
               wnll_ref, w_ref, correct_ref, l1_ref, nbox_ref):
    i = pl.program_id(0)

    @pl.when(i == 0)
    def _init():
        wnll_ref[0, 0] = 0.0
        w_ref[0, 0] = 0.0
        correct_ref[0, 0] = 0.0
        l1_ref[0, 0] = 0.0
        nbox_ref[0, 0] = jnp.sum(sizes_ref[...]).astype(jnp.float32)

    q = logits_ref.shape[1]
    c1 = logits_ref.shape[2]

    wnll_acc = 0.0
    w_acc = 0.0
    correct_acc = 0.0
    l1_acc = 0.0
    for j in range(nb):
        x = logits_ref[j]
        tgt = tgt_ref[pl.ds(j * s, s)]
        tc = tgt[:, 4:5].astype(jnp.int32)

        tc_full = jnp.concatenate(
            [tc, jnp.full((q - s, 1), num_classes, dtype=jnp.int32)], axis=0)

        m = jnp.max(x, axis=-1, keepdims=True)
        se = jnp.sum(jnp.exp(x - m), axis=-1, keepdims=True)
        lse = jnp.log(se) + m

        cols = lax.broadcasted_iota(jnp.int32, (q, c1), 1)
        sel = cols == tc_full
        x_tc = jnp.sum(jnp.where(sel, x, 0.0), axis=-1, keepdims=True)
        nll = lse - x_tc
        w = jnp.where(tc_full == num_classes, EOS_COEF, 1.0)
        wnll_acc += jnp.sum(w * nll)
        w_acc += jnp.sum(w)

        xm = x[:s]
        mm = jnp.max(xm, axis=-1, keepdims=True)
        cols_s = lax.broadcasted_iota(jnp.int32, (s, c1), 1)
        am = jnp.min(jnp.where(xm == mm, cols_s, c1), axis=-1, keepdims=True)
        correct_acc += jnp.sum((am == tc).astype(jnp.float32))

        pb = boxes_ref[j, :s, :]
        l1_acc += jnp.sum(jnp.abs(pb - tgt[:, 0:4]))

    wnll_ref[0, 0] += wnll_acc
    w_ref[0, 0] += w_acc
    correct_ref[0, 0] += correct_acc
    l1_ref[0, 0] += l1_acc


@jax.jit
def kernel(class_logits, pred_boxes, targets, sizes):
    b, q, c1 = class_logits.shape
    num_classes = c1 - 1
    s = targets.shape[0] // b

    nb = 4
    grid = (b // nb,)

    scalar = jax.ShapeDtypeStruct((1, 1), jnp.float32)
    smem_out = pl.BlockSpec(memory_space=pltpu.SMEM)
    sizes2d = sizes.reshape(8, b // 8)

    out = pl.pallas_call(
        functools.partial(_loss_body, nb, s, num_classes),
        grid=grid,
        in_specs=[
            pl.BlockSpec((nb, q, c1), lambda i: (i, 0, 0)),
            pl.BlockSpec((nb, q, 4), lambda i: (i, 0, 0)),
            pl.BlockSpec((nb * s, 5), lambda i: (i, 0)),
            pl.BlockSpec((8, b // 8), lambda i: (0, 0)),
        ],
        out_specs=[smem_out] * 5,
        out_shape=[scalar] * 5,
    )(class_logits, pred_boxes, targets, sizes2d)

    wnll, wsum, correct, l1, nbox = [o[0, 0] for o in out]
    loss_ce = wnll / wsum
    class_error = 100.0 - correct * (100.0 / (b * s))
    loss_bbox = l1 / jnp.maximum(nbox, 1.0)
    return loss_ce, class_error, loss_bbox


# speedup vs baseline: 4.7065x; 1.6056x over previous
import functools

import jax
import jax.numpy as jnp
from jax import lax
from jax.experimental import pallas as pl
from jax.experimental.pallas import tpu as pltpu

EOS_COEF = 0.1


def _loss_body(nb, s, num_classes, logits_ref, boxes_ref, tgt_ref, sizes_ref,
               wnll_ref, w_ref, correct_ref, l1_ref, nbox_ref):
    i = pl.program_id(0)

    @pl.when(i == 0)
    def _init():
        wnll_ref[0, 0] = 0.0
        w_ref[0, 0] = 0.0
        correct_ref[0, 0] = 0.0
        l1_ref[0, 0] = 0.0
        nbox_ref[0, 0] = jnp.sum(sizes_ref[...]).astype(jnp.float32)

    q = logits_ref.shape[1]
    c1 = logits_ref.shape[2]

    x = logits_ref[...]
    m = jnp.max(x, axis=-1, keepdims=True)
    se = jnp.sum(jnp.exp(x - m), axis=-1, keepdims=True)
    lse = jnp.log(se) + m

    dense_nll = jnp.sum(lse - x[:, :, c1 - 1:c1])

    xm = jnp.concatenate([x[j, :s] for j in range(nb)], axis=0)
    lse_m = jnp.concatenate([lse[j, :s] for j in range(nb)], axis=0)
    tgt = tgt_ref[...]
    tc = tgt[:, 4:5].astype(jnp.int32)

    cols = lax.broadcasted_iota(jnp.int32, (nb * s, c1), 1)
    x_tc = jnp.sum(jnp.where(cols == tc, xm, 0.0), axis=-1, keepdims=True)
    w = jnp.where(tc == num_classes, EOS_COEF, 1.0)
    x_last = xm[:, c1 - 1:c1]
    corr = jnp.sum(w * (lse_m - x_tc) - EOS_COEF * (lse_m - x_last))

    mm = jnp.max(xm, axis=-1, keepdims=True)
    am = jnp.min(jnp.where(xm == mm, cols, c1), axis=-1, keepdims=True)
    correct = jnp.sum((am == tc).astype(jnp.float32))

    pb = jnp.concatenate([boxes_ref[j, :s, :] for j in range(nb)], axis=0)
    l1 = jnp.sum(jnp.abs(pb - tgt[:, 0:4]))

    wnll_ref[0, 0] += EOS_COEF * dense_nll + corr
    w_ref[0, 0] += EOS_COEF * (nb * (q - s)) + jnp.sum(w)
    correct_ref[0, 0] += correct
    l1_ref[0, 0] += l1


@jax.jit
def kernel(class_logits, pred_boxes, targets, sizes):
    b, q, c1 = class_logits.shape
    num_classes = c1 - 1
    s = targets.shape[0] // b

    nb = 8
    grid = (b // nb,)

    scalar = jax.ShapeDtypeStruct((1, 1), jnp.float32)
    smem_out = pl.BlockSpec(memory_space=pltpu.SMEM)
    sizes2d = sizes.reshape(8, b // 8)

    out = pl.pallas_call(
        functools.partial(_loss_body, nb, s, num_classes),
        grid=grid,
        in_specs=[
            pl.BlockSpec((nb, q, c1), lambda i: (i, 0, 0)),
            pl.BlockSpec((nb, q, 4), lambda i: (i, 0, 0)),
            pl.BlockSpec((nb * s, 5), lambda i: (i, 0)),
            pl.BlockSpec((8, b // 8), lambda i: (0, 0)),
        ],
        out_specs=[smem_out] * 5,
        out_shape=[scalar] * 5,
    )(class_logits, pred_boxes, targets, sizes2d)

    wnll, wsum, correct, l1, nbox = [o[0, 0] for o in out]
    loss_ce = wnll / wsum
    class_error = 100.0 - correct * (100.0 / (b * s))
    loss_bbox = l1 / jnp.maximum(nbox, 1.0)
    return loss_ce, class_error, loss_bbox
